# baseline (speedup 1.0000x reference)
import jax
import jax.numpy as jnp
from jax.experimental import pallas as pl
from jax.experimental.pallas import tpu as pltpu


def _som_dist_kernel(x_ref, w_ref, out_ref):
    RB, C, D = w_ref.shape
    w = w_ref.reshape(RB * C, D)[...]
    x = x_ref[...]
    xs = x * -2.0
    xw = jax.lax.dot_general(
        xs, w, (((1,), (1,)), ((), ())),
        preferred_element_type=jnp.float32,
    )
    x2 = jnp.sum(x * x, axis=1, keepdims=True)
    ones = jnp.ones((1, D), jnp.float32)
    w2 = jax.lax.dot_general(
        ones, w * w, (((1,), (1,)), ((), ())),
        preferred_element_type=jnp.float32,
    )
    out_ref[...] = (xw + x2) + w2


def kernel(x, weights):
    R, C, D = weights.shape
    B = x.shape[0]
    N = R * C
    STEPS = 2
    out = pl.pallas_call(
        _som_dist_kernel,
        grid=(STEPS,),
        in_specs=[
            pl.BlockSpec((B, D), lambda i: (0, 0)),
            pl.BlockSpec((R // STEPS, C, D), lambda i: (i, 0, 0)),
        ],
        out_specs=pl.BlockSpec((B, N // STEPS), lambda i: (0, i)),
        out_shape=jax.ShapeDtypeStruct((B, N), jnp.float32),
        compiler_params=pltpu.CompilerParams(
            dimension_semantics=("arbitrary",),
        ),
    )(x, weights)
    return out.reshape(B, R, C)
